# Initial kernel scaffold; baseline (speedup 1.0000x reference)
import math

import jax
import jax.numpy as jnp
from jax import lax
from jax.experimental import pallas as pl
from jax.experimental.pallas import tpu as pltpu
from jax.experimental.pallas import tpu_sc as plsc

NUM_NODES = 10000
N_EDGES = 320000
D = 128
H = 4

LANES = 16
N_TILES = 16
E_TILE = N_EDGES // N_TILES
NSEG_ROWS = 640
BLK = 2000

_INV_SQRT_D = 1.0 / math.sqrt(D)
_INV_SQRT_2 = 1.0 / math.sqrt(2.0)


def _att_body(z_ref, wq_ref, wk_ref, out_ref, aqk_ref):
    @pl.when(pl.program_id(0) == 0)
    def _():
        for h in range(H):
            wq = wq_ref[:, h * D:(h + 1) * D]
            wk = wk_ref[:, h * D:(h + 1) * D]
            aqk_ref[:, h * D:(h + 1) * D] = lax.dot_general(
                wq, wk, (((1,), (1,)), ((), ())),
                preferred_element_type=jnp.float32)

    z = z_ref[...]
    p = jnp.dot(z, aqk_ref[...], preferred_element_type=jnp.float32)
    cols = [
        jnp.sum(p[:, h * D:(h + 1) * D] * z, axis=1, keepdims=True)
        * _INV_SQRT_D
        for h in range(H)
    ]
    out_ref[...] = jnp.concatenate(cols, axis=1)


def _softmax_body(att_hbm, row_hbm, w_hbm,
                  att0, att1, rowv, acc0, acc1, iota_v, s0_sh, s1_sh):
    cid = lax.axis_index("c")
    sid = lax.axis_index("s")
    base = sid * E_TILE
    off0 = (2 * cid) * N_EDGES + base
    off1 = (2 * cid + 1) * N_EDGES + base

    pltpu.sync_copy(row_hbm.at[pl.ds(base, E_TILE)], rowv)
    pltpu.sync_copy(att_hbm.at[pl.ds(off0, E_TILE)], att0)
    pltpu.sync_copy(att_hbm.at[pl.ds(off1, E_TILE)], att1)

    zero16 = jnp.zeros((LANES,), jnp.float32)

    def _zero(i, _):
        acc0[i, :] = zero16
        acc1[i, :] = zero16
        return 0

    lax.fori_loop(0, NSEG_ROWS, _zero, 0)

    rows_per_tile = NSEG_ROWS // N_TILES
    pltpu.sync_copy(acc0.at[pl.ds(0, rows_per_tile)],
                    s0_sh.at[pl.ds(sid * rows_per_tile, rows_per_tile)])
    pltpu.sync_copy(acc1.at[pl.ds(0, rows_per_tile)],
                    s1_sh.at[pl.ds(sid * rows_per_tile, rows_per_tile)])

    for k in range(NSEG_ROWS // 128):
        for i in range(128 // LANES):
            iota_v[k, pl.ds(i * LANES, LANES)] = (
                lax.iota(jnp.int32, LANES) + (k * 128 + i * LANES))

    plsc.subcore_barrier()

    n_vec = E_TILE // LANES

    def _pass1(i, _):
        sl = pl.ds(i * LANES, LANES)
        idx = rowv[sl]
        r = lax.shift_right_logical(idx, 4)
        c = lax.bitwise_and(idx, LANES - 1)
        e0 = jnp.exp(att0[sl])
        att0[sl] = e0
        plsc.addupdate_scatter(acc0, [r, c], e0)
        e1 = jnp.exp(att1[sl])
        att1[sl] = e1
        plsc.addupdate_scatter(acc1, [r, c], e1)
        return 0

    lax.fori_loop(0, n_vec, _pass1, 0)

    for k in range(NSEG_ROWS // 128):
        pltpu.sync_copy(acc0.at[pl.ds(k * 128, 128)],
                        s0_sh.at[iota_v.at[k]], add=True)
        pltpu.sync_copy(acc1.at[pl.ds(k * 128, 128)],
                        s1_sh.at[iota_v.at[k]], add=True)

    plsc.subcore_barrier()

    pltpu.sync_copy(s0_sh, acc0)
    pltpu.sync_copy(s1_sh, acc1)

    def _pass2(i, _):
        sl = pl.ds(i * LANES, LANES)
        idx = rowv[sl]
        r = lax.shift_right_logical(idx, 4)
        c = lax.bitwise_and(idx, LANES - 1)
        s0 = plsc.load_gather(acc0, [r, c])
        att0[sl] = att0[sl] / s0
        s1 = plsc.load_gather(acc1, [r, c])
        att1[sl] = att1[sl] / s1
        return 0

    lax.fori_loop(0, n_vec, _pass2, 0)

    pltpu.sync_copy(att0, w_hbm.at[pl.ds(off0, E_TILE)])
    pltpu.sync_copy(att1, w_hbm.at[pl.ds(off1, E_TILE)])


def _out_body(z_ref, w_ref, wv_ref, wg_ref, bg_ref, w1_ref, b1_ref,
              w2_ref, b2_ref, out_ref):
    z = z_ref[...]
    v = jnp.dot(z, wv_ref[...], preferred_element_type=jnp.float32)
    g = jax.nn.sigmoid(
        jnp.dot(z, wg_ref[...], preferred_element_type=jnp.float32)
        + bg_ref[...])
    vg = v * g
    w = w_ref[...]
    out1 = z
    for h in range(H):
        out1 = out1 + w[:, h:h + 1] * vg[:, h * D:(h + 1) * D]
    t = jnp.dot(out1, w1_ref[...], preferred_element_type=jnp.float32) \
        + b1_ref[...]
    ge = 0.5 * t * (1.0 + lax.erf(t * _INV_SQRT_2))
    out_ref[...] = out1 \
        + jnp.dot(ge, w2_ref[...], preferred_element_type=jnp.float32) \
        + b2_ref[...]


def kernel(Z, edges, Wq, Wk, Wv, Wg, bg, W1, b1, W2, b2):
    row = edges[0]

    wq_all = jnp.transpose(Wq, (1, 0, 2)).reshape(D, H * D)
    wk_all = jnp.transpose(Wk, (1, 0, 2)).reshape(D, H * D)
    wv_all = jnp.transpose(Wv, (1, 0, 2)).reshape(D, H * D)
    wg_all = jnp.transpose(Wg, (1, 0, 2)).reshape(D, H * D)
    bg_all = bg.reshape(1, H * D)

    att_eh = pl.pallas_call(
        _att_body,
        grid=(N_EDGES // BLK,),
        in_specs=[
            pl.BlockSpec((BLK, D), lambda i: (i, 0)),
            pl.BlockSpec((D, H * D), lambda i: (0, 0)),
            pl.BlockSpec((D, H * D), lambda i: (0, 0)),
        ],
        out_specs=pl.BlockSpec((BLK, H), lambda i: (i, 0)),
        out_shape=jax.ShapeDtypeStruct((N_EDGES, H), jnp.float32),
        scratch_shapes=[pltpu.VMEM((D, H * D), jnp.float32)],
    )(Z, wq_all, wk_all)

    att_flat = att_eh.T.reshape(H * N_EDGES)

    mesh = plsc.VectorSubcoreMesh(core_axis_name="c", subcore_axis_name="s")
    w_flat = pl.kernel(
        _softmax_body,
        out_type=jax.ShapeDtypeStruct((H * N_EDGES,), jnp.float32),
        mesh=mesh,
        scratch_types=[
            pltpu.VMEM((E_TILE,), jnp.float32),
            pltpu.VMEM((E_TILE,), jnp.float32),
            pltpu.VMEM((E_TILE,), jnp.int32),
            pltpu.VMEM((NSEG_ROWS, LANES), jnp.float32),
            pltpu.VMEM((NSEG_ROWS, LANES), jnp.float32),
            pltpu.VMEM((NSEG_ROWS // 128, 128), jnp.int32),
            pltpu.VMEM_SHARED((NSEG_ROWS, LANES), jnp.float32),
            pltpu.VMEM_SHARED((NSEG_ROWS, LANES), jnp.float32),
        ],
    )(att_flat, row)

    w_edge = w_flat.reshape(H, N_EDGES).T

    out = pl.pallas_call(
        _out_body,
        grid=(N_EDGES // BLK,),
        in_specs=[
            pl.BlockSpec((BLK, D), lambda i: (i, 0)),
            pl.BlockSpec((BLK, H), lambda i: (i, 0)),
            pl.BlockSpec((D, H * D), lambda i: (0, 0)),
            pl.BlockSpec((D, H * D), lambda i: (0, 0)),
            pl.BlockSpec((1, H * D), lambda i: (0, 0)),
            pl.BlockSpec((D, D), lambda i: (0, 0)),
            pl.BlockSpec((1, D), lambda i: (0, 0)),
            pl.BlockSpec((D, D), lambda i: (0, 0)),
            pl.BlockSpec((1, D), lambda i: (0, 0)),
        ],
        out_specs=pl.BlockSpec((BLK, D), lambda i: (i, 0)),
        out_shape=jax.ShapeDtypeStruct((N_EDGES, D), jnp.float32),
    )(Z, w_edge, wv_all, wg_all, bg_all, W1,
      b1.reshape(1, D), W2, b2.reshape(1, D))

    return out


# speedup vs baseline: 19.8180x; 19.8180x over previous
import math

import jax
import jax.numpy as jnp
from jax import lax
from jax.experimental import pallas as pl
from jax.experimental.pallas import tpu as pltpu
from jax.experimental.pallas import tpu_sc as plsc

NUM_NODES = 10000
N_EDGES = 320000
D = 128
H = 4

LANES = 16
N_TILES = 16
E_TILE = N_EDGES // N_TILES
NSEG_ROWS = 640
BLK = 2000

_INV_SQRT_D = 1.0 / math.sqrt(D)
_INV_SQRT_2 = 1.0 / math.sqrt(2.0)


def _att_body(z_ref, wq_ref, wk_ref, out_ref, aqk_ref):
    @pl.when(pl.program_id(0) == 0)
    def _():
        for h in range(H):
            wq = wq_ref[:, h * D:(h + 1) * D]
            wk = wk_ref[:, h * D:(h + 1) * D]
            aqk_ref[:, h * D:(h + 1) * D] = lax.dot_general(
                wq, wk, (((1,), (1,)), ((), ())),
                preferred_element_type=jnp.float32)

    z = z_ref[...]
    p = jnp.dot(z, aqk_ref[...], preferred_element_type=jnp.float32)
    cols = [
        jnp.sum(p[:, h * D:(h + 1) * D] * z, axis=1, keepdims=True)
        * _INV_SQRT_D
        for h in range(H)
    ]
    out_ref[...] = jnp.concatenate(cols, axis=1)


NSEG = NSEG_ROWS * LANES


def _softmax_body(att_hbm, row_hbm, w_hbm,
                  att0, att1, rowv, acc0, acc1, tmp_v, sum_v,
                  st0_sh, st1_sh, f0_sh, f1_sh):
    cid = lax.axis_index("c")
    sid = lax.axis_index("s")
    base = sid * E_TILE
    off0 = (2 * cid) * N_EDGES + base
    off1 = (2 * cid + 1) * N_EDGES + base

    pltpu.sync_copy(row_hbm.at[pl.ds(base, E_TILE)], rowv)
    pltpu.sync_copy(att_hbm.at[pl.ds(off0, E_TILE)], att0)
    pltpu.sync_copy(att_hbm.at[pl.ds(off1, E_TILE)], att1)

    zero16 = jnp.zeros((LANES,), jnp.float32)

    def _zero(i, _):
        acc0[pl.ds(i * LANES, LANES)] = zero16
        acc1[pl.ds(i * LANES, LANES)] = zero16
        return 0

    lax.fori_loop(0, NSEG // LANES, _zero, 0)

    n_vec = E_TILE // LANES

    def _pass1(i, _):
        sl = pl.ds(i * LANES, LANES)
        idx = rowv[sl]
        e0 = jnp.exp(att0[sl])
        att0[sl] = e0
        plsc.addupdate_scatter(acc0, [idx], e0)
        e1 = jnp.exp(att1[sl])
        att1[sl] = e1
        plsc.addupdate_scatter(acc1, [idx], e1)
        return 0

    lax.fori_loop(0, n_vec, _pass1, 0)

    pltpu.sync_copy(acc0, st0_sh.at[sid])
    pltpu.sync_copy(acc1, st1_sh.at[sid])
    plsc.subcore_barrier()

    nseg_tile = NSEG // N_TILES
    seg_base = sid * nseg_tile

    for j, (st_sh, f_sh) in enumerate(((st0_sh, f0_sh), (st1_sh, f1_sh))):
        def _zs(i, _):
            sum_v[pl.ds(i * LANES, LANES)] = zero16
            return 0
        lax.fori_loop(0, nseg_tile // LANES, _zs, 0)
        for t in range(N_TILES):
            pltpu.sync_copy(st_sh.at[t, pl.ds(seg_base, nseg_tile)], tmp_v)

            def _acc(i, _):
                sl = pl.ds(i * LANES, LANES)
                sum_v[sl] = sum_v[sl] + tmp_v[sl]
                return 0

            lax.fori_loop(0, nseg_tile // LANES, _acc, 0)
        pltpu.sync_copy(sum_v, f_sh.at[pl.ds(seg_base, nseg_tile)])

    plsc.subcore_barrier()

    pltpu.sync_copy(f0_sh, acc0)
    pltpu.sync_copy(f1_sh, acc1)

    def _pass2(i, _):
        sl = pl.ds(i * LANES, LANES)
        idx = rowv[sl]
        s0 = plsc.load_gather(acc0, [idx])
        att0[sl] = att0[sl] / s0
        s1 = plsc.load_gather(acc1, [idx])
        att1[sl] = att1[sl] / s1
        return 0

    lax.fori_loop(0, n_vec, _pass2, 0)

    pltpu.sync_copy(att0, w_hbm.at[pl.ds(off0, E_TILE)])
    pltpu.sync_copy(att1, w_hbm.at[pl.ds(off1, E_TILE)])


def _out_body(z_ref, w_ref, wv_ref, wg_ref, bg_ref, w1_ref, b1_ref,
              w2_ref, b2_ref, out_ref):
    z = z_ref[...]
    v = jnp.dot(z, wv_ref[...], preferred_element_type=jnp.float32)
    g = jax.nn.sigmoid(
        jnp.dot(z, wg_ref[...], preferred_element_type=jnp.float32)
        + bg_ref[...])
    vg = v * g
    w = w_ref[...]
    out1 = z
    for h in range(H):
        out1 = out1 + w[:, h:h + 1] * vg[:, h * D:(h + 1) * D]
    t = jnp.dot(out1, w1_ref[...], preferred_element_type=jnp.float32) \
        + b1_ref[...]
    ge = 0.5 * t * (1.0 + lax.erf(t * _INV_SQRT_2))
    out_ref[...] = out1 \
        + jnp.dot(ge, w2_ref[...], preferred_element_type=jnp.float32) \
        + b2_ref[...]


def kernel(Z, edges, Wq, Wk, Wv, Wg, bg, W1, b1, W2, b2):
    row = edges[0]

    wq_all = jnp.transpose(Wq, (1, 0, 2)).reshape(D, H * D)
    wk_all = jnp.transpose(Wk, (1, 0, 2)).reshape(D, H * D)
    wv_all = jnp.transpose(Wv, (1, 0, 2)).reshape(D, H * D)
    wg_all = jnp.transpose(Wg, (1, 0, 2)).reshape(D, H * D)
    bg_all = bg.reshape(1, H * D)

    att_eh = pl.pallas_call(
        _att_body,
        grid=(N_EDGES // BLK,),
        in_specs=[
            pl.BlockSpec((BLK, D), lambda i: (i, 0)),
            pl.BlockSpec((D, H * D), lambda i: (0, 0)),
            pl.BlockSpec((D, H * D), lambda i: (0, 0)),
        ],
        out_specs=pl.BlockSpec((BLK, H), lambda i: (i, 0)),
        out_shape=jax.ShapeDtypeStruct((N_EDGES, H), jnp.float32),
        scratch_shapes=[pltpu.VMEM((D, H * D), jnp.float32)],
    )(Z, wq_all, wk_all)

    att_flat = att_eh.T.reshape(H * N_EDGES)

    mesh = plsc.VectorSubcoreMesh(core_axis_name="c", subcore_axis_name="s")
    w_flat = pl.kernel(
        _softmax_body,
        out_type=jax.ShapeDtypeStruct((H * N_EDGES,), jnp.float32),
        mesh=mesh,
        compiler_params=pltpu.CompilerParams(needs_layout_passes=False),
        scratch_types=[
            pltpu.VMEM((E_TILE,), jnp.float32),
            pltpu.VMEM((E_TILE,), jnp.float32),
            pltpu.VMEM((E_TILE,), jnp.int32),
            pltpu.VMEM((NSEG,), jnp.float32),
            pltpu.VMEM((NSEG,), jnp.float32),
            pltpu.VMEM((NSEG // N_TILES,), jnp.float32),
            pltpu.VMEM((NSEG // N_TILES,), jnp.float32),
            pltpu.VMEM_SHARED((N_TILES, NSEG), jnp.float32),
            pltpu.VMEM_SHARED((N_TILES, NSEG), jnp.float32),
            pltpu.VMEM_SHARED((NSEG,), jnp.float32),
            pltpu.VMEM_SHARED((NSEG,), jnp.float32),
        ],
    )(att_flat, row)

    w_edge = w_flat.reshape(H, N_EDGES).T

    out = pl.pallas_call(
        _out_body,
        grid=(N_EDGES // BLK,),
        in_specs=[
            pl.BlockSpec((BLK, D), lambda i: (i, 0)),
            pl.BlockSpec((BLK, H), lambda i: (i, 0)),
            pl.BlockSpec((D, H * D), lambda i: (0, 0)),
            pl.BlockSpec((D, H * D), lambda i: (0, 0)),
            pl.BlockSpec((1, H * D), lambda i: (0, 0)),
            pl.BlockSpec((D, D), lambda i: (0, 0)),
            pl.BlockSpec((1, D), lambda i: (0, 0)),
            pl.BlockSpec((D, D), lambda i: (0, 0)),
            pl.BlockSpec((1, D), lambda i: (0, 0)),
        ],
        out_specs=pl.BlockSpec((BLK, D), lambda i: (i, 0)),
        out_shape=jax.ShapeDtypeStruct((N_EDGES, D), jnp.float32),
    )(Z, w_edge, wv_all, wg_all, bg_all, W1,
      b1.reshape(1, D), W2, b2.reshape(1, D))

    return out


# speedup vs baseline: 24.7192x; 1.2473x over previous
import math

import jax
import jax.numpy as jnp
from jax import lax
from jax.experimental import pallas as pl
from jax.experimental.pallas import tpu as pltpu
from jax.experimental.pallas import tpu_sc as plsc

NUM_NODES = 10000
N_EDGES = 320000
D = 128
H = 4

LANES = 16
N_TILES = 16
E_TILE = N_EDGES // N_TILES
NSEG_ROWS = 640
BLK = 2560

_INV_SQRT_D = 1.0 / math.sqrt(D)
_INV_SQRT_2 = 1.0 / math.sqrt(2.0)


def _att_body(z_ref, wq_ref, wk_ref, out_ref, aqk_ref, hsum_ref):
    @pl.when(pl.program_id(0) == 0)
    def _():
        for h in range(H):
            wq = wq_ref[:, h * D:(h + 1) * D]
            wk = wk_ref[:, h * D:(h + 1) * D]
            aqk_ref[:, h * D:(h + 1) * D] = lax.dot_general(
                wq, wk, (((1,), (1,)), ((), ())),
                preferred_element_type=jnp.float32).astype(jnp.bfloat16)
        r = lax.broadcasted_iota(jnp.int32, (H * D, 8), 0) // D
        c = lax.broadcasted_iota(jnp.int32, (H * D, 8), 1)
        hsum_ref[...] = jnp.where(r == c, _INV_SQRT_D, 0.0).astype(
            jnp.bfloat16)

    z = z_ref[...]
    zb = z.astype(jnp.bfloat16)
    p = jnp.dot(zb, aqk_ref[...], preferred_element_type=jnp.float32)
    q = (p * jnp.concatenate([z] * H, axis=1)).astype(jnp.bfloat16)
    att_b8 = jnp.dot(q, hsum_ref[...], preferred_element_type=jnp.float32)
    out_ref[...] = att_b8.T


NSEG = NSEG_ROWS * LANES


def _softmax_body(att_hbm, row_hbm, w_hbm,
                  att0, att1, rowv, acc0, acc1, tmp_v, sum_v,
                  st0_sh, st1_sh, f0_sh, f1_sh):
    cid = lax.axis_index("c")
    sid = lax.axis_index("s")
    base = sid * E_TILE
    off0 = (2 * cid) * N_EDGES + base
    off1 = (2 * cid + 1) * N_EDGES + base

    pltpu.sync_copy(row_hbm.at[pl.ds(base, E_TILE)], rowv)
    pltpu.sync_copy(att_hbm.at[pl.ds(off0, E_TILE)], att0)
    pltpu.sync_copy(att_hbm.at[pl.ds(off1, E_TILE)], att1)

    zero16 = jnp.zeros((LANES,), jnp.float32)

    def _zero(i, _):
        acc0[pl.ds(i * LANES, LANES)] = zero16
        acc1[pl.ds(i * LANES, LANES)] = zero16
        return 0

    lax.fori_loop(0, NSEG // LANES, _zero, 0)

    n_vec = E_TILE // LANES

    def _pass1(i, _):
        sl = pl.ds(i * LANES, LANES)
        idx = rowv[sl]
        e0 = jnp.exp(att0[sl])
        att0[sl] = e0
        plsc.addupdate_scatter(acc0, [idx], e0)
        e1 = jnp.exp(att1[sl])
        att1[sl] = e1
        plsc.addupdate_scatter(acc1, [idx], e1)
        return 0

    lax.fori_loop(0, n_vec, _pass1, 0)

    pltpu.sync_copy(acc0, st0_sh.at[sid])
    pltpu.sync_copy(acc1, st1_sh.at[sid])
    plsc.subcore_barrier()

    nseg_tile = NSEG // N_TILES
    seg_base = sid * nseg_tile

    for j, (st_sh, f_sh) in enumerate(((st0_sh, f0_sh), (st1_sh, f1_sh))):
        def _zs(i, _):
            sum_v[pl.ds(i * LANES, LANES)] = zero16
            return 0
        lax.fori_loop(0, nseg_tile // LANES, _zs, 0)
        for t in range(N_TILES):
            pltpu.sync_copy(st_sh.at[t, pl.ds(seg_base, nseg_tile)], tmp_v)

            def _acc(i, _):
                sl = pl.ds(i * LANES, LANES)
                sum_v[sl] = sum_v[sl] + tmp_v[sl]
                return 0

            lax.fori_loop(0, nseg_tile // LANES, _acc, 0)
        pltpu.sync_copy(sum_v, f_sh.at[pl.ds(seg_base, nseg_tile)])

    plsc.subcore_barrier()

    pltpu.sync_copy(f0_sh, acc0)
    pltpu.sync_copy(f1_sh, acc1)

    def _pass2(i, _):
        sl = pl.ds(i * LANES, LANES)
        idx = rowv[sl]
        s0 = plsc.load_gather(acc0, [idx])
        att0[sl] = att0[sl] / s0
        s1 = plsc.load_gather(acc1, [idx])
        att1[sl] = att1[sl] / s1
        return 0

    lax.fori_loop(0, n_vec, _pass2, 0)

    pltpu.sync_copy(att0, w_hbm.at[pl.ds(off0, E_TILE)])
    pltpu.sync_copy(att1, w_hbm.at[pl.ds(off1, E_TILE)])


def _out_body(z_ref, w_ref, wv_ref, wg_ref, bg_ref, w1_ref, b1_ref,
              w2_ref, b2_ref, out_ref):
    z = z_ref[...]
    zb = z.astype(jnp.bfloat16)
    v = jnp.dot(zb, wv_ref[...], preferred_element_type=jnp.float32)
    g = jax.nn.sigmoid(
        jnp.dot(zb, wg_ref[...], preferred_element_type=jnp.float32)
        + bg_ref[...])
    vg = v * g
    wt = w_ref[...].T
    out1 = z
    for h in range(H):
        out1 = out1 + wt[:, h:h + 1] * vg[:, h * D:(h + 1) * D]
    t = jnp.dot(out1.astype(jnp.bfloat16), w1_ref[...],
                preferred_element_type=jnp.float32) + b1_ref[...]
    ge = 0.5 * t * (1.0 + lax.erf(t * _INV_SQRT_2))
    out_ref[...] = out1 \
        + jnp.dot(ge.astype(jnp.bfloat16), w2_ref[...],
                  preferred_element_type=jnp.float32) + b2_ref[...]


def kernel(Z, edges, Wq, Wk, Wv, Wg, bg, W1, b1, W2, b2):
    row = edges[0]

    wq_all = jnp.transpose(Wq, (1, 0, 2)).reshape(D, H * D)
    wk_all = jnp.transpose(Wk, (1, 0, 2)).reshape(D, H * D)
    wv_all = jnp.transpose(Wv, (1, 0, 2)).reshape(D, H * D).astype(
        jnp.bfloat16)
    wg_all = jnp.transpose(Wg, (1, 0, 2)).reshape(D, H * D).astype(
        jnp.bfloat16)
    bg_all = bg.reshape(1, H * D)

    att8 = pl.pallas_call(
        _att_body,
        grid=(N_EDGES // BLK,),
        in_specs=[
            pl.BlockSpec((BLK, D), lambda i: (i, 0)),
            pl.BlockSpec((D, H * D), lambda i: (0, 0)),
            pl.BlockSpec((D, H * D), lambda i: (0, 0)),
        ],
        out_specs=pl.BlockSpec((8, BLK), lambda i: (0, i)),
        out_shape=jax.ShapeDtypeStruct((8, N_EDGES), jnp.float32),
        scratch_shapes=[pltpu.VMEM((D, H * D), jnp.bfloat16),
                        pltpu.VMEM((H * D, 8), jnp.bfloat16)],
    )(Z, wq_all, wk_all)

    att_flat = att8.reshape(8 * N_EDGES)

    mesh = plsc.VectorSubcoreMesh(core_axis_name="c", subcore_axis_name="s")
    w_flat = pl.kernel(
        _softmax_body,
        out_type=jax.ShapeDtypeStruct((8 * N_EDGES,), jnp.float32),
        mesh=mesh,
        compiler_params=pltpu.CompilerParams(needs_layout_passes=False),
        scratch_types=[
            pltpu.VMEM((E_TILE,), jnp.float32),
            pltpu.VMEM((E_TILE,), jnp.float32),
            pltpu.VMEM((E_TILE,), jnp.int32),
            pltpu.VMEM((NSEG,), jnp.float32),
            pltpu.VMEM((NSEG,), jnp.float32),
            pltpu.VMEM((NSEG // N_TILES,), jnp.float32),
            pltpu.VMEM((NSEG // N_TILES,), jnp.float32),
            pltpu.VMEM_SHARED((N_TILES, NSEG), jnp.float32),
            pltpu.VMEM_SHARED((N_TILES, NSEG), jnp.float32),
            pltpu.VMEM_SHARED((NSEG,), jnp.float32),
            pltpu.VMEM_SHARED((NSEG,), jnp.float32),
        ],
    )(att_flat, row)

    w8 = w_flat.reshape(8, N_EDGES)

    out = pl.pallas_call(
        _out_body,
        grid=(N_EDGES // BLK,),
        in_specs=[
            pl.BlockSpec((BLK, D), lambda i: (i, 0)),
            pl.BlockSpec((8, BLK), lambda i: (0, i)),
            pl.BlockSpec((D, H * D), lambda i: (0, 0)),
            pl.BlockSpec((D, H * D), lambda i: (0, 0)),
            pl.BlockSpec((1, H * D), lambda i: (0, 0)),
            pl.BlockSpec((D, D), lambda i: (0, 0)),
            pl.BlockSpec((1, D), lambda i: (0, 0)),
            pl.BlockSpec((D, D), lambda i: (0, 0)),
            pl.BlockSpec((1, D), lambda i: (0, 0)),
        ],
        out_specs=pl.BlockSpec((BLK, D), lambda i: (i, 0)),
        out_shape=jax.ShapeDtypeStruct((N_EDGES, D), jnp.float32),
    )(Z, w8, wv_all, wg_all, bg_all, W1.astype(jnp.bfloat16),
      b1.reshape(1, D), W2.astype(jnp.bfloat16), b2.reshape(1, D))

    return out
